# Initial kernel scaffold; baseline (speedup 1.0000x reference)
import jax
import jax.numpy as jnp
from jax.experimental import pallas as pl


def kernel(x, edge_index, edge_weight, W1, b1, W2, b2):
    raise NotImplementedError("write your pallas kernel here")


# speedup vs baseline: 2.2401x; 2.2401x over previous
import functools

import jax
import jax.numpy as jnp
from jax import lax
from jax.experimental import pallas as pl
from jax.experimental.pallas import tpu as pltpu
from jax.experimental.pallas import tpu_sc as plsc

N = 10000
NP = 10240
E = 320000
F_IN = 128
H = 128
C = 64
CK = 3200
BN = 1024


def _stage_a_body(x_ref, w_ref, out_ref):
    out_ref[...] = lax.dot_general(
        w_ref[...], x_ref[...], (((0,), (1,)), ((), ())),
        preferred_element_type=jnp.float32)


def _stage_a(xp, W1):
    return pl.pallas_call(
        _stage_a_body,
        grid=(NP // BN,),
        in_specs=[pl.BlockSpec((BN, F_IN), lambda i: (i, 0)),
                  pl.BlockSpec((F_IN, H), lambda i: (0, 0))],
        out_specs=pl.BlockSpec((H, BN), lambda i: (0, i)),
        out_shape=jax.ShapeDtypeStruct((H, NP), jnp.float32),
    )(xp, W1)


def _stage_b_body(acc_ref, b1_ref, w2_ref, out_ref):
    a = jnp.maximum(acc_ref[...] + b1_ref[...], 0.0)
    out_ref[...] = lax.dot_general(
        w2_ref[...], a, (((0,), (0,)), ((), ())),
        preferred_element_type=jnp.float32)


def _stage_b(acc1T, b1c, W2):
    return pl.pallas_call(
        _stage_b_body,
        grid=(NP // BN,),
        in_specs=[pl.BlockSpec((H, BN), lambda i: (0, i)),
                  pl.BlockSpec((H, 1), lambda i: (0, 0)),
                  pl.BlockSpec((H, C), lambda i: (0, 0))],
        out_specs=pl.BlockSpec((C, BN), lambda i: (0, i)),
        out_shape=jax.ShapeDtypeStruct((C, NP), jnp.float32),
    )(acc1T, b1c, W2)


def _stage_c_body(acc_ref, b2_ref, out_ref):
    z = acc_ref[...] + b2_ref[...]
    m = jnp.max(z, axis=0, keepdims=True)
    lse = jnp.log(jnp.sum(jnp.exp(z - m), axis=0, keepdims=True)) + m
    out_ref[...] = (z - lse).T


def _stage_c(acc2T, b2c):
    return pl.pallas_call(
        _stage_c_body,
        grid=(NP // BN,),
        in_specs=[pl.BlockSpec((C, BN), lambda i: (0, i)),
                  pl.BlockSpec((C, 1), lambda i: (0, 0))],
        out_specs=pl.BlockSpec((BN, C), lambda i: (i, 0)),
        out_shape=jax.ShapeDtypeStruct((NP, C), jnp.float32),
    )(acc2T, b2c)


@functools.cache
def _make_spmm(F, FPT):
    assert 32 * FPT == F
    info = plsc.get_sparse_core_info()
    nc = info.num_cores
    mesh = plsc.VectorSubcoreMesh(core_axis_name="c", subcore_axis_name="s")

    def body(hT, srcH, dstH, wH, outH, hrows, acc, srcb, dstb, wb):
        fg = lax.axis_index("s") * nc + lax.axis_index("c")
        f0 = fg * FPT
        pltpu.sync_copy(hT.at[pl.ds(f0, FPT)], hrows)

        z16 = jnp.zeros((16,), jnp.float32)

        def zero_body(i, _):
            for f in range(FPT):
                acc[f, pl.ds(i * 16, 16)] = z16
            return 0

        lax.fori_loop(0, NP // 16, zero_body, 0)

        def chunk_body(ci, _):
            base = ci * CK
            pltpu.sync_copy(srcH.at[pl.ds(base, CK)], srcb)
            pltpu.sync_copy(dstH.at[pl.ds(base, CK)], dstb)
            pltpu.sync_copy(wH.at[pl.ds(base, CK)], wb)

            def group_body(g, _):
                o = g * 16
                s16 = srcb[pl.ds(o, 16)]
                d16 = dstb[pl.ds(o, 16)]
                w16 = wb[pl.ds(o, 16)]
                for f in range(FPT):
                    f16 = jnp.full((16,), f, jnp.int32)
                    vals = plsc.load_gather(hrows, [f16, s16])
                    plsc.addupdate_scatter(acc, [f16, d16], vals * w16)
                return 0

            lax.fori_loop(0, CK // 16, group_body, 0)
            return 0

        lax.fori_loop(0, E // CK, chunk_body, 0)
        pltpu.sync_copy(acc, outH.at[pl.ds(f0, FPT)])

    return pl.kernel(
        body,
        out_type=jax.ShapeDtypeStruct((F, NP), jnp.float32),
        mesh=mesh,
        compiler_params=pltpu.CompilerParams(
            use_tc_tiling_on_sc=False, needs_layout_passes=False),
        scratch_types=[
            pltpu.VMEM((FPT, NP), jnp.float32),
            pltpu.VMEM((FPT, NP), jnp.float32),
            pltpu.VMEM((CK,), jnp.int32),
            pltpu.VMEM((CK,), jnp.int32),
            pltpu.VMEM((CK,), jnp.float32),
        ],
    )


@jax.jit
def kernel(x, edge_index, edge_weight, W1, b1, W2, b2):
    src = edge_index[1]
    dst = edge_index[0]
    xp = jnp.pad(x, ((0, NP - N), (0, 0)))
    h1T = _stage_a(xp, W1)
    acc1T = _make_spmm(H, H // 32)(h1T, src, dst, edge_weight)
    h2T = _stage_b(acc1T, b1.reshape(H, 1), W2)
    acc2T = _make_spmm(C, C // 32)(h2T, src, dst, edge_weight)
    out = _stage_c(acc2T, b2.reshape(C, 1))
    return out[:N]


# speedup vs baseline: 3.8431x; 1.7156x over previous
import functools

import jax
import jax.numpy as jnp
from jax import lax
from jax.experimental import pallas as pl
from jax.experimental.pallas import tpu as pltpu
from jax.experimental.pallas import tpu_sc as plsc

N = 10000
NP = 10240
E = 320000
F_IN = 128
H = 128
C = 64
CK = 3200
BN = 1024


def _stage_a_body(x_ref, w_ref, out_ref):
    out_ref[...] = lax.dot_general(
        w_ref[...], x_ref[...], (((0,), (1,)), ((), ())),
        preferred_element_type=jnp.float32)


def _stage_a(xp, W1):
    return pl.pallas_call(
        _stage_a_body,
        grid=(NP // BN,),
        in_specs=[pl.BlockSpec((BN, F_IN), lambda i: (i, 0)),
                  pl.BlockSpec((F_IN, H), lambda i: (0, 0))],
        out_specs=pl.BlockSpec((H, BN), lambda i: (0, i)),
        out_shape=jax.ShapeDtypeStruct((H, NP), jnp.float32),
    )(xp, W1)


def _stage_b_body(acc_ref, b1_ref, w2_ref, out_ref):
    a = jnp.maximum(acc_ref[...] + b1_ref[...], 0.0)
    out_ref[...] = lax.dot_general(
        w2_ref[...], a, (((0,), (0,)), ((), ())),
        preferred_element_type=jnp.float32)


def _stage_b(acc1T, b1c, W2):
    return pl.pallas_call(
        _stage_b_body,
        grid=(NP // BN,),
        in_specs=[pl.BlockSpec((H, BN), lambda i: (0, i)),
                  pl.BlockSpec((H, 1), lambda i: (0, 0)),
                  pl.BlockSpec((H, C), lambda i: (0, 0))],
        out_specs=pl.BlockSpec((C, BN), lambda i: (0, i)),
        out_shape=jax.ShapeDtypeStruct((C, NP), jnp.float32),
    )(acc1T, b1c, W2)


def _stage_c_body(acc_ref, b2_ref, out_ref):
    z = acc_ref[...] + b2_ref[...]
    m = jnp.max(z, axis=0, keepdims=True)
    lse = jnp.log(jnp.sum(jnp.exp(z - m), axis=0, keepdims=True)) + m
    out_ref[...] = (z - lse).T


def _stage_c(acc2T, b2c):
    return pl.pallas_call(
        _stage_c_body,
        grid=(NP // BN,),
        in_specs=[pl.BlockSpec((C, BN), lambda i: (0, i)),
                  pl.BlockSpec((C, 1), lambda i: (0, 0))],
        out_specs=pl.BlockSpec((BN, C), lambda i: (i, 0)),
        out_shape=jax.ShapeDtypeStruct((NP, C), jnp.float32),
    )(acc2T, b2c)


@functools.cache
def _make_spmm(F, FPT):
    assert 32 * FPT == F
    info = plsc.get_sparse_core_info()
    nc = info.num_cores
    mesh = plsc.VectorSubcoreMesh(core_axis_name="c", subcore_axis_name="s")

    def body(hT, srcH, dstH, wH, outH, hrows, acc, srcb, dstb, wb):
        fg = lax.axis_index("s") * nc + lax.axis_index("c")
        f0 = fg * FPT
        pltpu.sync_copy(hT.at[pl.ds(f0, FPT)], hrows)

        z16 = jnp.zeros((16,), jnp.float32)

        @plsc.parallel_loop(0, NP, 16, unroll=8)
        def zero_body(i):
            for f in range(FPT):
                acc[f, pl.ds(i, 16)] = z16

        def chunk_body(ci, _):
            base = ci * CK
            pltpu.sync_copy(srcH.at[pl.ds(base, CK)], srcb)
            pltpu.sync_copy(dstH.at[pl.ds(base, CK)], dstb)
            pltpu.sync_copy(wH.at[pl.ds(base, CK)], wb)

            @plsc.parallel_loop(0, CK, 16, unroll=4)
            def group_body(o):
                s16 = srcb[pl.ds(o, 16)]
                d16 = dstb[pl.ds(o, 16)]
                w16 = wb[pl.ds(o, 16)]
                for f in range(FPT):
                    f16 = jnp.full((16,), f, jnp.int32)
                    vals = plsc.load_gather(hrows, [f16, s16])
                    plsc.addupdate_scatter(acc, [f16, d16], vals * w16)

            return 0

        lax.fori_loop(0, E // CK, chunk_body, 0)
        pltpu.sync_copy(acc, outH.at[pl.ds(f0, FPT)])

    return pl.kernel(
        body,
        out_type=jax.ShapeDtypeStruct((F, NP), jnp.float32),
        mesh=mesh,
        compiler_params=pltpu.CompilerParams(
            use_tc_tiling_on_sc=False, needs_layout_passes=False),
        scratch_types=[
            pltpu.VMEM((FPT, NP), jnp.float32),
            pltpu.VMEM((FPT, NP), jnp.float32),
            pltpu.VMEM((CK,), jnp.int32),
            pltpu.VMEM((CK,), jnp.int32),
            pltpu.VMEM((CK,), jnp.float32),
        ],
    )


@jax.jit
def kernel(x, edge_index, edge_weight, W1, b1, W2, b2):
    src = edge_index[1]
    dst = edge_index[0]
    xp = jnp.pad(x, ((0, NP - N), (0, 0)))
    h1T = _stage_a(xp, W1)
    acc1T = _make_spmm(H, H // 32)(h1T, src, dst, edge_weight)
    h2T = _stage_b(acc1T, b1.reshape(H, 1), W2)
    acc2T = _make_spmm(C, C // 32)(h2T, src, dst, edge_weight)
    out = _stage_c(acc2T, b2.reshape(C, 1))
    return out[:N]


# speedup vs baseline: 7.6386x; 1.9876x over previous
import functools

import jax
import jax.numpy as jnp
from jax import lax
from jax.experimental import pallas as pl
from jax.experimental.pallas import tpu as pltpu
from jax.experimental.pallas import tpu_sc as plsc

N = 10000
NP = 10240
E = 320000
F_IN = 128
H = 128
C = 64
CK = 3200
BN = 1024


def _stage_a_body(x_ref, w_ref, out_ref):
    out_ref[...] = lax.dot_general(
        w_ref[...], x_ref[...], (((0,), (1,)), ((), ())),
        preferred_element_type=jnp.float32)


def _stage_a(xp, W1):
    return pl.pallas_call(
        _stage_a_body,
        grid=(NP // BN,),
        in_specs=[pl.BlockSpec((BN, F_IN), lambda i: (i, 0)),
                  pl.BlockSpec((F_IN, H), lambda i: (0, 0))],
        out_specs=pl.BlockSpec((H, BN), lambda i: (0, i)),
        out_shape=jax.ShapeDtypeStruct((H, NP), jnp.float32),
    )(xp, W1)


def _stage_b_body(acc_ref, b1_ref, w2_ref, out_ref):
    a = jnp.maximum(acc_ref[...] + b1_ref[...], 0.0)
    out_ref[...] = lax.dot_general(
        w2_ref[...], a, (((0,), (0,)), ((), ())),
        preferred_element_type=jnp.float32)


def _stage_b(acc1T, b1c, W2):
    return pl.pallas_call(
        _stage_b_body,
        grid=(NP // BN,),
        in_specs=[pl.BlockSpec((H, BN), lambda i: (0, i)),
                  pl.BlockSpec((H, 1), lambda i: (0, 0)),
                  pl.BlockSpec((H, C), lambda i: (0, 0))],
        out_specs=pl.BlockSpec((C, BN), lambda i: (0, i)),
        out_shape=jax.ShapeDtypeStruct((C, NP), jnp.float32),
    )(acc1T, b1c, W2)


def _stage_c_body(acc_ref, b2_ref, out_ref):
    z = acc_ref[...] + b2_ref[...]
    m = jnp.max(z, axis=0, keepdims=True)
    lse = jnp.log(jnp.sum(jnp.exp(z - m), axis=0, keepdims=True)) + m
    out_ref[...] = (z - lse).T


def _stage_c(acc2T, b2c):
    return pl.pallas_call(
        _stage_c_body,
        grid=(NP // BN,),
        in_specs=[pl.BlockSpec((C, BN), lambda i: (0, i)),
                  pl.BlockSpec((C, 1), lambda i: (0, 0))],
        out_specs=pl.BlockSpec((BN, C), lambda i: (i, 0)),
        out_shape=jax.ShapeDtypeStruct((NP, C), jnp.float32),
    )(acc2T, b2c)


@functools.cache
def _make_spmm(F, FPT):
    assert 32 * FPT == F
    info = plsc.get_sparse_core_info()
    nc = info.num_cores
    mesh = plsc.VectorSubcoreMesh(core_axis_name="c", subcore_axis_name="s")

    NCH = E // CK
    assert NCH % 2 == 0

    def body(hT, srcH, dstH, wH, outH, hrows, acc, srcb, dstb, wb, sem):
        fg = lax.axis_index("s") * nc + lax.axis_index("c")
        f0 = fg * FPT
        pltpu.sync_copy(hT.at[pl.ds(f0, FPT)], hrows)

        z16 = jnp.zeros((16,), jnp.float32)

        @plsc.parallel_loop(0, NP, 16, unroll=8)
        def zero_body(i):
            for f in range(FPT):
                acc[f, pl.ds(i, 16)] = z16

        def issue(ci, b):
            base = ci * CK
            pltpu.async_copy(srcH.at[pl.ds(base, CK)], srcb.at[b], sem.at[b])
            pltpu.async_copy(dstH.at[pl.ds(base, CK)], dstb.at[b], sem.at[b])
            pltpu.async_copy(wH.at[pl.ds(base, CK)], wb.at[b], sem.at[b])

        def wait(b):
            pltpu.make_async_copy(srcH.at[pl.ds(0, CK)], srcb.at[b],
                                  sem.at[b]).wait()
            pltpu.make_async_copy(dstH.at[pl.ds(0, CK)], dstb.at[b],
                                  sem.at[b]).wait()
            pltpu.make_async_copy(wH.at[pl.ds(0, CK)], wb.at[b],
                                  sem.at[b]).wait()

        issue(0, 0)

        def pair_body(p, _):
            for b in range(2):
                ci = p * 2 + b

                @pl.when(ci + 1 < NCH)
                def _():
                    issue(ci + 1, 1 - b)

                wait(b)

                @plsc.parallel_loop(0, CK, 16, unroll=4)
                def group_body(o):
                    s16 = srcb[b, pl.ds(o, 16)]
                    d16 = dstb[b, pl.ds(o, 16)]
                    w16 = wb[b, pl.ds(o, 16)]
                    for f in range(FPT):
                        f16 = jnp.full((16,), f, jnp.int32)
                        vals = plsc.load_gather(hrows, [f16, s16])
                        plsc.addupdate_scatter(acc, [f16, d16], vals * w16)

            return 0

        lax.fori_loop(0, NCH // 2, pair_body, 0)
        pltpu.sync_copy(acc, outH.at[pl.ds(f0, FPT)])

    return pl.kernel(
        body,
        out_type=jax.ShapeDtypeStruct((F, NP), jnp.float32),
        mesh=mesh,
        compiler_params=pltpu.CompilerParams(
            use_tc_tiling_on_sc=False, needs_layout_passes=False),
        scratch_types=[
            pltpu.VMEM((FPT, NP), jnp.float32),
            pltpu.VMEM((FPT, NP), jnp.float32),
            pltpu.VMEM((2, CK), jnp.int32),
            pltpu.VMEM((2, CK), jnp.int32),
            pltpu.VMEM((2, CK), jnp.float32),
            pltpu.SemaphoreType.DMA((2,)),
        ],
    )


@jax.jit
def kernel(x, edge_index, edge_weight, W1, b1, W2, b2):
    src = edge_index[1]
    dst = edge_index[0]
    xp = jnp.pad(x, ((0, NP - N), (0, 0)))
    h1T = _stage_a(xp, W1)
    acc1T = _make_spmm(H, H // 32)(h1T, src, dst, edge_weight)
    h2T = _stage_b(acc1T, b1.reshape(H, 1), W2)
    acc2T = _make_spmm(C, C // 32)(h2T, src, dst, edge_weight)
    out = _stage_c(acc2T, b2.reshape(C, 1))
    return out[:N]
